# baseline (speedup 1.0000x reference)
import functools

import jax
import jax.numpy as jnp
from jax import lax
from jax.experimental import pallas as pl
from jax.experimental.pallas import tpu as pltpu
from jax.experimental.pallas import tpu_sc as plsc

NC, NS, L = 2, 16, 16
NW = NC * NS
B, SEQ, C = 4096, 200, 64
V = 1_000_000
RCH = 256
NCHUNK = V // RCH
TAIL = V - NCHUNK * RCH
BPW = B // NW
NOCT = SEQ // 8


def _mesh():
    return plsc.VectorSubcoreMesh(
        core_axis_name="c", subcore_axis_name="s",
        num_cores=NC, num_subcores=NS)


def _make_k1():
    @functools.partial(
        pl.kernel,
        out_type=jax.ShapeDtypeStruct((V, 128), jnp.float32),
        mesh=_mesh(),
        scratch_types=[
            pltpu.VMEM((2, C, RCH), jnp.float32),
            pltpu.VMEM((2, RCH, 128), jnp.float32),
            pltpu.VMEM((C, TAIL), jnp.float32),
            pltpu.VMEM((TAIL, 128), jnp.float32),
            pltpu.SemaphoreType.DMA((2,)),
            pltpu.SemaphoreType.DMA((2,)),
        ],
        compiler_params=pltpu.CompilerParams(
            use_tc_tiling_on_sc=True, needs_layout_passes=False),
    )
    def k1(tt_hbm, tail_hbm, trm_hbm, tbuf, obuf, tailin, tailout,
           in_sem, out_sem):
        wid = lax.axis_index("s") * NC + lax.axis_index("c")
        niter = (NCHUNK - wid + NW - 1) // NW
        ciota = lax.iota(jnp.int32, L)

        def r0_of(i):
            return pl.multiple_of((wid + i * NW) * RCH, RCH)

        def start_in(i, p):
            pltpu.async_copy(
                tt_hbm.at[:, pl.ds(r0_of(i), RCH)], tbuf.at[p], in_sem.at[p])

        def wait_in(p):
            pltpu.make_async_copy(
                tt_hbm.at[:, pl.ds(0, RCH)], tbuf.at[p], in_sem.at[p]).wait()

        def start_out(i, p):
            pltpu.async_copy(
                obuf.at[p], trm_hbm.at[pl.ds(r0_of(i), RCH)], out_sem.at[p])

        def wait_out(p):
            pltpu.make_async_copy(
                obuf.at[p], trm_hbm.at[pl.ds(0, RCH)], out_sem.at[p]).wait()

        @pl.when(wid == 0)
        def _():
            pltpu.sync_copy(tail_hbm, tailin)

            @plsc.parallel_loop(0, TAIL, 1, unroll=4)
            def tbody(r):
                rsplat = jnp.full((L,), r, jnp.int32)
                for k in range(C // L):
                    v = plsc.load_gather(tailin, [ciota + k * L, rsplat])
                    tailout[r, pl.ds(k * L, L)] = v

            pltpu.sync_copy(tailout, trm_hbm.at[pl.ds(V - TAIL, TAIL)])

        start_in(0, 0)

        def body(i, carry):
            p = lax.rem(i, 2)

            @pl.when(i + 1 < niter)
            def _():
                start_in(i + 1, 1 - p)

            wait_in(p)

            @pl.when(i >= 2)
            def _():
                wait_out(p)

            tb = tbuf.at[p]

            @plsc.parallel_loop(0, RCH, 1, unroll=4)
            def rbody(r):
                rsplat = jnp.full((L,), r, jnp.int32)
                for k in range(C // L):
                    v = plsc.load_gather(tb, [ciota + k * L, rsplat])
                    obuf[p, r, pl.ds(k * L, L)] = v

            start_out(i, p)
            return carry

        lax.fori_loop(0, niter, body, 0)
        wait_out(lax.rem(niter - 1, 2))
        wait_out(lax.rem(niter, 2))

    return k1


def _make_k2():
    @functools.partial(
        pl.kernel,
        out_type=jax.ShapeDtypeStruct((C, SEQ, B), jnp.float32),
        mesh=_mesh(),
        scratch_types=[
            pltpu.VMEM((2, 8, 128), jnp.int32),
            pltpu.VMEM((2, 8, 128), jnp.float32),
            pltpu.VMEM((128,), jnp.int32),
            pltpu.VMEM((128,), jnp.int32),
            pltpu.VMEM((3, 128, 128), jnp.float32),
            pltpu.VMEM((C, 8, 128), jnp.float32),
            pltpu.SemaphoreType.DMA((2,)),
            pltpu.SemaphoreType.DMA((3,)),
            pltpu.SemaphoreType.DMA,
        ],
        compiler_params=pltpu.CompilerParams(
            use_tc_tiling_on_sc=True, needs_layout_passes=False),
    )
    def k2(xt_hbm, mt_hbm, trm_hbm, out_hbm,
           idx_v, mask_v, idx1d_a, idx1d_b, rows_v, outt_v,
           in_sem, gat_sem, out_sem):
        wid = lax.axis_index("s") * NC + lax.axis_index("c")
        b0 = pl.multiple_of(wid * BPW, 128)
        biota = lax.iota(jnp.int32, L)

        def l0_of(j):
            return pl.multiple_of(j * 8, 8)

        def start_in(j, p):
            pltpu.async_copy(
                xt_hbm.at[pl.ds(l0_of(j), 8), pl.ds(b0, 128)], idx_v.at[p],
                in_sem.at[p])
            pltpu.async_copy(
                mt_hbm.at[pl.ds(l0_of(j), 8), pl.ds(b0, 128)], mask_v.at[p],
                in_sem.at[p])

        def wait_in(p):
            pltpu.make_async_copy(
                xt_hbm.at[pl.ds(0, 8), pl.ds(0, 128)], idx_v.at[p],
                in_sem.at[p]).wait()
            pltpu.make_async_copy(
                mt_hbm.at[pl.ds(0, 8), pl.ds(0, 128)], mask_v.at[p],
                in_sem.at[p]).wait()

        def issue_gather(pj, t):
            slot = lax.rem(t, 3)

            def _do(buf):
                for k in range(128 // L):
                    buf[pl.ds(k * L, L)] = idx_v[pj, t, pl.ds(k * L, L)]
                pltpu.async_copy(
                    trm_hbm.at[buf], rows_v.at[slot], gat_sem.at[slot])

            @pl.when(lax.rem(t, 2) == 0)
            def _():
                _do(idx1d_a)

            @pl.when(lax.rem(t, 2) == 1)
            def _():
                _do(idx1d_b)

        def wait_gather(slot):
            pltpu.make_async_copy(
                trm_hbm.at[pl.ds(0, 128)], rows_v.at[slot],
                gat_sem.at[slot]).wait()

        def wait_out():
            pltpu.make_async_copy(
                outt_v, out_hbm.at[:, pl.ds(0, 8), pl.ds(0, 128)],
                out_sem).wait()

        start_in(0, 0)
        start_in(1, 1)
        wait_in(0)
        issue_gather(0, 0)

        def octet(j, carry):
            pj = lax.rem(j, 2)

            def lbody(dl, carry2):
                wait_gather(lax.rem(dl, 3))

                @pl.when(dl == 0)
                def _():
                    issue_gather(pj, 1)

                @pl.when(dl < 6)
                def _():
                    issue_gather(pj, dl + 2)

                @pl.when((dl == 7) & (j < NOCT - 1))
                def _():
                    wait_in(1 - pj)
                    issue_gather(1 - pj, 0)

                rows = rows_v.at[lax.rem(dl, 3)]
                mvs = [mask_v[pj, dl, pl.ds(k * L, L)] for k in range(8)]
                lidx = [biota + k * L for k in range(8)]

                @pl.when((j > 0) & (dl == 0))
                def _():
                    wait_out()

                @plsc.parallel_loop(0, C, 1, unroll=2)
                def cbody(c):
                    csplat = jnp.full((L,), c, jnp.int32)
                    for k in range(8):
                        v = plsc.load_gather(rows, [lidx[k], csplat])
                        outt_v[c, dl, pl.ds(k * L, L)] = v * mvs[k]

                return carry2

            lax.fori_loop(0, 8, lbody, 0)

            @pl.when(j < NOCT - 2)
            def _():
                start_in(j + 2, pj)

            pltpu.async_copy(
                outt_v,
                out_hbm.at[:, pl.ds(l0_of(j), 8), pl.ds(b0, 128)], out_sem)
            return carry

        lax.fori_loop(0, NOCT, octet, 0)
        wait_out()

    return k2


_k1 = _make_k1()
_k2 = _make_k2()


@jax.jit
def kernel(x, mask, table):
    xt = x.astype(jnp.int32).T
    mt = mask.reshape(B, SEQ).T
    tt = table.T
    ttail = lax.slice(table, (V - TAIL, 0), (V, C)).T
    trm = _k1(tt, ttail)
    outt = _k2(xt, mt, trm)
    return outt.transpose(2, 0, 1)
